# baseline (speedup 1.0000x reference)
import functools

import jax
import jax.numpy as jnp
from jax import lax
from jax.experimental import pallas as pl
from jax.experimental.pallas import tpu as pltpu
from jax.experimental.pallas import tpu_sc as plsc

_BETA = 0.2
_L = 16
_NC = 2
_NS = 16
_NW = _NC * _NS
_K = 64
_CPP = 125


def _sc_update(P, D, N, m2, idx, val, out2,
               idx_v, winner_v, wj_v, stage_v, sj_v, rows_m, rows_v1, rows_v2,
               cbuf0, cbuf1, seml0, seml1, sems0, sems1, sem_m, sem_1, sem_2):
    PW = P // _NW
    R = 2 * PW
    wid = lax.axis_index("s") * _NC + lax.axis_index("c")
    pbase = wid * PW
    base = 2 * pbase
    lane = lax.iota(jnp.int32, _L)

    ncp = PW // _CPP

    def crange(c):
        return pl.ds(pbase + c * _CPP, _CPP)

    npairs = ncp // 2

    def cpair(it, carry):
        c0 = it * 2
        l0 = pltpu.async_copy(m2.at[crange(c0)], cbuf0, seml0)
        l1 = pltpu.async_copy(m2.at[crange(c0 + 1)], cbuf1, seml1)
        l0.wait()
        s0 = pltpu.async_copy(cbuf0, out2.at[crange(c0)], sems0)
        l1.wait()
        s1 = pltpu.async_copy(cbuf1, out2.at[crange(c0 + 1)], sems1)
        s0.wait()
        s1.wait()
        return carry

    lax.fori_loop(0, npairs, cpair, jnp.int32(0))
    if ncp % 2:
        c0 = ncp - 1
        pltpu.async_copy(m2.at[crange(c0)], cbuf0, seml0).wait()
        pltpu.async_copy(cbuf0, out2.at[crange(c0)], sems0).wait()

    WPAD = (R + _L - 1) // _L
    neg1 = jnp.full((_L,), -1, jnp.int32)

    def winit(c, carry):
        winner_v[pl.ds(c * _L, _L)] = neg1
        return carry

    lax.fori_loop(0, WPAD, winit, jnp.int32(0))

    pltpu.sync_copy(idx, idx_v)

    nch = N // _L

    def p1(c, carry):
        jv = idx_v[pl.ds(c * _L, _L)]
        pos = lane + c * _L
        loc = jv - base
        m = (loc >= 0) & (loc < R)
        locc = jnp.where(m, loc, 0)
        _, lastm = plsc.scan_count(jv, mask=m)
        plsc.store_scatter(winner_v, [locc], pos, mask=m & lastm)
        return carry

    lax.fori_loop(0, nch, p1, jnp.int32(0))

    def p2(c, cnt):
        jv = idx_v[pl.ds(c * _L, _L)]
        pos = lane + c * _L
        loc = jv - base
        m = (loc >= 0) & (loc < R)
        locc = jnp.where(m, loc, 0)
        rb = plsc.load_gather(winner_v, [locc], mask=m)
        win = m & (rb == pos)
        sib = plsc.load_gather(winner_v, [locc ^ 1], mask=m)
        h = jv & 1
        lead = win & ((h == 0) | (sib < 0))
        plsc.store_compressed(wj_v.at[pl.ds(cnt, _L)], pos, mask=lead)
        return cnt + jnp.sum(lead.astype(jnp.int32))

    cnt = lax.fori_loop(0, nch, p2, jnp.int32(0))

    lastp = jnp.full((_L,), 0, jnp.int32) + jnp.maximum(cnt - 1, 0)
    lastj = plsc.load_gather(wj_v, [lastp])

    def pad(q, carry):
        wj_v[pl.ds(cnt + q * _L, _L)] = lastj
        return carry

    lax.fori_loop(0, _K // _L, pad, jnp.int32(0))

    nt = (cnt + _K - 1) // _K

    def p3(t, carry):
        off = t * _K

        def st(q, c2):
            wjq = wj_v[pl.ds(off + q * _L, _L)]
            iq = plsc.load_gather(idx_v, [wjq])
            pq = iq >> 1
            eloc = 2 * pq - base
            jev = plsc.load_gather(winner_v, [eloc])
            jov = plsc.load_gather(winner_v, [eloc + 1])
            s = pl.ds(q * _L, _L)
            stage_v[0, s] = pq
            stage_v[1, s] = jnp.maximum(jev, 0)
            stage_v[2, s] = jnp.maximum(jov, 0)
            sj_v[0, s] = jev
            sj_v[1, s] = jov
            return c2

        lax.fori_loop(0, _K // _L, st, jnp.int32(0), unroll=True)

        gm = pltpu.async_copy(m2.at[stage_v.at[0]], rows_m, sem_m)
        g1 = pltpu.async_copy(val.at[stage_v.at[1]], rows_v1, sem_1)
        g2 = pltpu.async_copy(val.at[stage_v.at[2]], rows_v2, sem_2)
        gm.wait()
        g1.wait()
        g2.wait()

        def bgroup(g, c2):
            rvec = g * _L + lane
            s = pl.ds(g * _L, _L)
            me = sj_v[0, s] >= 0
            mo = sj_v[1, s] >= 0

            def bcol(c, c3):
                cv = jnp.full((_L,), 0, jnp.int32) + c
                a = plsc.load_gather(rows_m, [rvec, cv])
                v = plsc.load_gather(rows_v1, [rvec, cv])
                plsc.store_scatter(rows_m, [rvec, cv],
                                   a * _BETA + v * (1.0 - _BETA), mask=me)
                a2 = plsc.load_gather(rows_m, [rvec, cv + (D // 2)])
                v2 = plsc.load_gather(rows_v2, [rvec, cv])
                plsc.store_scatter(rows_m, [rvec, cv + (D // 2)],
                                   a2 * _BETA + v2 * (1.0 - _BETA), mask=mo)
                return c3

            lax.fori_loop(0, D // 2, bcol, jnp.int32(0))
            return c2

        lax.fori_loop(0, _K // _L, bgroup, jnp.int32(0))
        pltpu.sync_copy(rows_m, out2.at[stage_v.at[0]])
        return carry

    lax.fori_loop(0, nt, p3, jnp.int32(0))


def kernel(mem, idx, val):
    M, Dm = mem.shape
    N = idx.shape[0]
    P, D = M // 2, Dm * 2
    assert P % _NW == 0 and (P // _NW) % _CPP == 0
    assert N % _L == 0 and Dm % _L == 0

    mesh = plsc.VectorSubcoreMesh(core_axis_name="c", subcore_axis_name="s")
    cap = N + _K + _L
    wpad = ((2 * P // _NW + _L - 1) // _L) * _L
    run = pl.kernel(
        functools.partial(_sc_update, P, D, N),
        out_type=jax.ShapeDtypeStruct((P, D), jnp.float32),
        mesh=mesh,
        compiler_params=pltpu.CompilerParams(use_tc_tiling_on_sc=False,
                                             needs_layout_passes=False),
        scratch_types=[
            pltpu.VMEM((N,), jnp.int32),
            pltpu.VMEM((wpad,), jnp.int32),
            pltpu.VMEM((cap,), jnp.int32),
            pltpu.VMEM((3, _K), jnp.int32),
            pltpu.VMEM((2, _K), jnp.int32),
            pltpu.VMEM((_K, D), jnp.float32),
            pltpu.VMEM((_K, Dm), jnp.float32),
            pltpu.VMEM((_K, Dm), jnp.float32),
            pltpu.VMEM((_CPP, D), jnp.float32),
            pltpu.VMEM((_CPP, D), jnp.float32),
            pltpu.SemaphoreType.DMA,
            pltpu.SemaphoreType.DMA,
            pltpu.SemaphoreType.DMA,
            pltpu.SemaphoreType.DMA,
            pltpu.SemaphoreType.DMA,
            pltpu.SemaphoreType.DMA,
            pltpu.SemaphoreType.DMA,
        ],
    )
    m2 = mem.reshape(P, D)
    out2 = run(m2, idx.astype(jnp.int32), val)
    return out2.reshape(M, Dm)
